# Initial kernel scaffold; baseline (speedup 1.0000x reference)
import jax
import jax.numpy as jnp
from jax.experimental import pallas as pl


def kernel(x, batch, y, W_h, b_h, W_o, b_o):
    raise NotImplementedError("write your pallas kernel here")


# speedup vs baseline: 11.3700x; 11.3700x over previous
import functools

import jax
import jax.numpy as jnp
from jax.experimental import pallas as pl
from jax.experimental.pallas import tpu as pltpu

NUM_GRAPHS = 1024
WIDTH = 512
HIDDEN = 64
NUM_CLASSES = 2
N_NODES = 50000

BLOCK = 2000
NBLK = N_NODES // BLOCK


def _kernel(x_ref, batch_ref, y_ref, wh_ref, bh_ref, wo_ref, bo_ref,
            target_ref, preds_ref, acc_ref):
    i = pl.program_id(0)

    @pl.when(i == 0)
    def _():
        acc_ref[...] = jnp.zeros_like(acc_ref)

    x = x_ref[...]
    seg = batch_ref[0, 0, :]

    p = jax.lax.dot_general(
        x, wh_ref[...],
        dimension_numbers=(((1,), (1,)), ((), ())),
        preferred_element_type=jnp.float32)

    ones = jnp.ones((BLOCK, 1), dtype=jnp.float32)
    p_aug = jnp.concatenate([p, ones], axis=1)

    gids = jax.lax.broadcasted_iota(jnp.int32, (NUM_GRAPHS, BLOCK), 0)
    onehot = (gids == seg[None, :]).astype(jnp.float32)
    acc_ref[...] += jax.lax.dot_general(
        onehot, p_aug,
        dimension_numbers=(((1,), (0,)), ((), ())),
        preferred_element_type=jnp.float32)

    @pl.when(i == NBLK - 1)
    def _():
        sums = acc_ref[:, :HIDDEN]
        counts = acc_ref[:, HIDDEN:HIDDEN + 1]
        emb_h = sums / jnp.maximum(counts, 1.0)
        h = jnp.maximum(emb_h + bh_ref[...], 0.0)
        preds = jax.lax.dot_general(
            h, wo_ref[...],
            dimension_numbers=(((1,), (1,)), ((), ())),
            preferred_element_type=jnp.float32) + bo_ref[...]
        m = jnp.max(preds, axis=1, keepdims=True)
        lse = m + jnp.log(jnp.sum(jnp.exp(preds - m), axis=1, keepdims=True))
        logp = preds - lse
        y = y_ref[...]
        loss = jnp.where(y == 0, -logp[:, 0:1], -logp[:, 1:2])
        target_ref[...] = (jnp.sum(loss) / NUM_GRAPHS).reshape(1, 1)
        preds_ref[...] = preds


@jax.jit
def kernel(x, batch, y, W_h, b_h, W_o, b_o):
    batch3 = batch.astype(jnp.int32).reshape(NBLK, 1, BLOCK)
    y2 = y.reshape(NUM_GRAPHS, 1)
    bh2 = b_h.reshape(1, HIDDEN)
    bo2 = b_o.reshape(1, NUM_CLASSES)

    target, preds = pl.pallas_call(
        _kernel,
        grid=(NBLK,),
        in_specs=[
            pl.BlockSpec((BLOCK, WIDTH), lambda i: (i, 0)),
            pl.BlockSpec((1, 1, BLOCK), lambda i: (i, 0, 0)),
            pl.BlockSpec((NUM_GRAPHS, 1), lambda i: (0, 0)),
            pl.BlockSpec((HIDDEN, WIDTH), lambda i: (0, 0)),
            pl.BlockSpec((1, HIDDEN), lambda i: (0, 0)),
            pl.BlockSpec((NUM_CLASSES, HIDDEN), lambda i: (0, 0)),
            pl.BlockSpec((1, NUM_CLASSES), lambda i: (0, 0)),
        ],
        out_specs=[
            pl.BlockSpec((1, 1), lambda i: (0, 0)),
            pl.BlockSpec((NUM_GRAPHS, NUM_CLASSES), lambda i: (0, 0)),
        ],
        out_shape=[
            jax.ShapeDtypeStruct((1, 1), jnp.float32),
            jax.ShapeDtypeStruct((NUM_GRAPHS, NUM_CLASSES), jnp.float32),
        ],
        scratch_shapes=[pltpu.VMEM((NUM_GRAPHS, HIDDEN + 1), jnp.float32)],
    )(x, batch3, y2, W_h, bh2, W_o, bo2)

    return (target[0, 0], preds)
